# baseline (speedup 1.0000x reference)
import jax
import jax.numpy as jnp
from jax.experimental import pallas as pl
from jax.experimental.pallas import tpu as pltpu

NUM_EXPERTS = 64
TOP_K = 8
D_MODEL = 4096
TOKENS = 16384

T_BLK = 512
N_BLOCKS = TOKENS // T_BLK
NBUF = 5


def _compute_block(xblk, w_ref, b_ref):
    logits = jax.lax.dot_general(
        xblk, w_ref[...],
        dimension_numbers=(((1,), (1,)), ((), ())),
        preferred_element_type=jnp.float32,
    )

    lt = logits.T + b_ref[...]
    zpart = jnp.sum(lt * lt, axis=1, keepdims=True)

    m = jnp.max(lt, axis=0, keepdims=True)
    e = jnp.exp(lt - m)
    s = jnp.sum(e, axis=0, keepdims=True)
    probs = e / s
    ppart = jnp.sum(probs, axis=1, keepdims=True)

    sub = jax.lax.broadcasted_iota(jnp.int32, probs.shape, 0)
    vals = probs
    ws = []
    idxs = []
    for _ in range(TOP_K):
        mk = jnp.max(vals, axis=0, keepdims=True)
        is_mk = vals >= mk
        idx = jnp.min(
            jnp.where(is_mk, sub, NUM_EXPERTS), axis=0, keepdims=True
        )
        ws.append(mk)
        idxs.append(idx)
        vals = jnp.where(sub == idx, -1.0, vals)

    w_cat = jnp.concatenate(ws, axis=0)
    wsum = jnp.sum(w_cat, axis=0, keepdims=True)
    wn = (w_cat / (wsum + 1e-8)).T
    ic = jnp.concatenate(idxs, axis=0).T
    return wn, ic, ppart, zpart


def _router(x_hbm, w_ref, b_ref, w_out, i_out, lbl_out, zl_out,
            xbuf, psum_acc, zsum_acc, sems):
    def start_copy(blk, b):
        pltpu.make_async_copy(
            x_hbm.at[pl.ds(blk * T_BLK, T_BLK), :],
            xbuf.at[b],
            sems.at[b],
        ).start()

    def wait_copy(blk, b):
        pltpu.make_async_copy(
            x_hbm.at[pl.ds(blk * T_BLK, T_BLK), :],
            xbuf.at[b],
            sems.at[b],
        ).wait()

    for b in range(NBUF):
        start_copy(b, b)

    def body(blk, carry):
        b = jax.lax.rem(blk, NBUF)
        wait_copy(blk, b)
        wn, ic, ppart, zpart = _compute_block(xbuf[b], w_ref, b_ref)
        w_out[pl.ds(blk * T_BLK, T_BLK), :] = wn
        i_out[pl.ds(blk * T_BLK, T_BLK), :] = ic

        @pl.when(blk == 0)
        def _init():
            psum_acc[...] = ppart
            zsum_acc[...] = zpart

        @pl.when(blk != 0)
        def _accum():
            psum_acc[...] += ppart
            zsum_acc[...] += zpart

        @pl.when(blk + NBUF < N_BLOCKS)
        def _next():
            start_copy(blk + NBUF, b)

        return carry

    jax.lax.fori_loop(0, N_BLOCKS, body, 0)

    tpe = psum_acc[...] / TOKENS
    u = 1.0 / NUM_EXPERTS
    lbl_out[0, 0] = jnp.sum((tpe - u) ** 2) * NUM_EXPERTS
    zl_out[0, 0] = jnp.sum(zsum_acc[...]) / (TOKENS * NUM_EXPERTS) * 0.001


@jax.jit
def kernel(x, W, expert_bias):
    bias = expert_bias.reshape(NUM_EXPERTS, 1)

    w_out, i_out, lbl, zl = pl.pallas_call(
        _router,
        in_specs=[
            pl.BlockSpec(memory_space=pl.ANY),
            pl.BlockSpec(memory_space=pltpu.VMEM),
            pl.BlockSpec(memory_space=pltpu.VMEM),
        ],
        out_specs=[
            pl.BlockSpec(memory_space=pltpu.VMEM),
            pl.BlockSpec(memory_space=pltpu.VMEM),
            pl.BlockSpec(memory_space=pltpu.SMEM),
            pl.BlockSpec(memory_space=pltpu.SMEM),
        ],
        out_shape=[
            jax.ShapeDtypeStruct((TOKENS, TOP_K), jnp.float32),
            jax.ShapeDtypeStruct((TOKENS, TOP_K), jnp.int32),
            jax.ShapeDtypeStruct((1, 1), jnp.float32),
            jax.ShapeDtypeStruct((1, 1), jnp.float32),
        ],
        scratch_shapes=[
            pltpu.VMEM((NBUF, T_BLK, D_MODEL), jnp.float32),
            pltpu.VMEM((NUM_EXPERTS, 1), jnp.float32),
            pltpu.VMEM((NUM_EXPERTS, 1), jnp.float32),
            pltpu.SemaphoreType.DMA((NBUF,)),
        ],
    )(x, W, bias)

    return (w_out, i_out, lbl.reshape(()), zl.reshape(()))


# speedup vs baseline: 1.0499x; 1.0499x over previous
import jax
import jax.numpy as jnp
from jax.experimental import pallas as pl
from jax.experimental.pallas import tpu as pltpu

NUM_EXPERTS = 64
TOP_K = 8
D_MODEL = 4096
TOKENS = 16384

T_BLK = 1024


def _router_block(x_ref, w_ref, b_ref, w_out, i_out, lbl_out, zl_out,
                  psum_acc, zsum_acc):
    step = pl.program_id(0)
    nsteps = pl.num_programs(0)

    logits = jax.lax.dot_general(
        x_ref[...], w_ref[...],
        dimension_numbers=(((1,), (1,)), ((), ())),
        preferred_element_type=jnp.float32,
    )

    lt = logits.T + b_ref[...]

    zpart = jnp.sum(lt * lt, axis=1, keepdims=True)

    m = jnp.max(lt, axis=0, keepdims=True)
    e = jnp.exp(lt - m)
    s = jnp.sum(e, axis=0, keepdims=True)
    probs = e / s

    ppart = jnp.sum(probs, axis=1, keepdims=True)

    @pl.when(step == 0)
    def _init():
        psum_acc[...] = ppart
        zsum_acc[...] = zpart

    @pl.when(step != 0)
    def _accum():
        psum_acc[...] += ppart
        zsum_acc[...] += zpart

    @pl.when(step == nsteps - 1)
    def _finalize():
        tpe = psum_acc[...] / TOKENS
        u = 1.0 / NUM_EXPERTS
        lbl_out[0, 0] = jnp.sum((tpe - u) ** 2) * NUM_EXPERTS
        zl_out[0, 0] = jnp.sum(zsum_acc[...]) / (TOKENS * NUM_EXPERTS) * 0.001

    sub = jax.lax.broadcasted_iota(jnp.int32, probs.shape, 0)
    vals = probs
    ws = []
    idxs = []
    for _ in range(TOP_K):
        mk = jnp.max(vals, axis=0, keepdims=True)
        is_mk = vals >= mk
        idx = jnp.min(
            jnp.where(is_mk, sub, NUM_EXPERTS), axis=0, keepdims=True
        )
        ws.append(mk)
        idxs.append(idx)
        vals = jnp.where(sub == idx, -1.0, vals)

    w_cat = jnp.concatenate(ws, axis=0)
    wsum = jnp.sum(w_cat, axis=0, keepdims=True)
    w_out[...] = (w_cat / (wsum + 1e-8)).T
    i_out[...] = jnp.concatenate(idxs, axis=0).T


@jax.jit
def kernel(x, W, expert_bias):
    grid = TOKENS // T_BLK
    bias = expert_bias.reshape(NUM_EXPERTS, 1)

    w_out, i_out, lbl, zl = pl.pallas_call(
        _router_block,
        grid=(grid,),
        in_specs=[
            pl.BlockSpec((T_BLK, D_MODEL), lambda i: (i, 0)),
            pl.BlockSpec((NUM_EXPERTS, D_MODEL), lambda i: (0, 0)),
            pl.BlockSpec((NUM_EXPERTS, 1), lambda i: (0, 0)),
        ],
        out_specs=[
            pl.BlockSpec((T_BLK, TOP_K), lambda i: (i, 0)),
            pl.BlockSpec((T_BLK, TOP_K), lambda i: (i, 0)),
            pl.BlockSpec(memory_space=pltpu.SMEM),
            pl.BlockSpec(memory_space=pltpu.SMEM),
        ],
        out_shape=[
            jax.ShapeDtypeStruct((TOKENS, TOP_K), jnp.float32),
            jax.ShapeDtypeStruct((TOKENS, TOP_K), jnp.int32),
            jax.ShapeDtypeStruct((1, 1), jnp.float32),
            jax.ShapeDtypeStruct((1, 1), jnp.float32),
        ],
        scratch_shapes=[
            pltpu.VMEM((NUM_EXPERTS, 1), jnp.float32),
            pltpu.VMEM((NUM_EXPERTS, 1), jnp.float32),
        ],
        compiler_params=pltpu.CompilerParams(
            dimension_semantics=("arbitrary",),
        ),
    )(x, W, bias)

    return (w_out, i_out, lbl.reshape(()), zl.reshape(()))
